# baseline (speedup 1.0000x reference)
import functools

import jax
import jax.numpy as jnp
from jax import lax
from jax.experimental import pallas as pl
from jax.experimental.pallas import tpu as pltpu
from jax.experimental.pallas import tpu_sc as plsc


def _copy_body(src, dst):
    dst[...] = src[...]


def _tc_copy(x, max_grid=20):
    rows, cols = x.shape
    g = 1
    for cand in range(max_grid, 0, -1):
        if rows % cand == 0 and (rows // cand) % 8 == 0:
            g = cand
            break
    b = rows // g
    return pl.pallas_call(
        _copy_body,
        grid=(g,),
        in_specs=(pl.BlockSpec((b, cols), lambda i: (i, 0)),),
        out_specs=pl.BlockSpec((b, cols), lambda i: (i, 0)),
        out_shape=jax.ShapeDtypeStruct((rows, cols), x.dtype),
    )(x)


def _sc_copy(x, chunk_target=2000):
    rows, cols = x.shape
    nw = 32
    if rows % nw != 0:
        return None
    rows_w = rows // nw

    chunk = 0
    for cand in range(min(chunk_target, rows_w), 0, -1):
        if rows_w % cand == 0 and cand % 8 == 0:
            chunk = cand
            break
    if chunk == 0 or rows_w % 8 != 0:
        return None
    nchunks = rows_w // chunk

    mesh = plsc.VectorSubcoreMesh(core_axis_name="c", subcore_axis_name="s")

    @functools.partial(
        pl.kernel,
        out_type=jax.ShapeDtypeStruct((rows, cols), x.dtype),
        mesh=mesh,
        compiler_params=pltpu.CompilerParams(use_tc_tiling_on_sc=False),
        scratch_types=[
            pltpu.VMEM((chunk, cols), x.dtype),
            pltpu.VMEM((chunk, cols), x.dtype),
            pltpu.SemaphoreType.DMA,
            pltpu.SemaphoreType.DMA,
            pltpu.SemaphoreType.DMA,
            pltpu.SemaphoreType.DMA,
        ],
    )
    def sc_copy(src_hbm, dst_hbm, buf0, buf1, lsem0, lsem1, ssem0, ssem1):
        wid = lax.axis_index("s") * 2 + lax.axis_index("c")
        base = wid * rows_w
        bufs = (buf0, buf1)
        lsems = (lsem0, lsem1)
        ssems = (ssem0, ssem1)

        def load(k):
            return pltpu.make_async_copy(
                src_hbm.at[pl.ds(base + k * chunk, chunk)],
                bufs[k % 2],
                lsems[k % 2],
            )

        def store(k):
            return pltpu.make_async_copy(
                bufs[k % 2],
                dst_hbm.at[pl.ds(base + k * chunk, chunk)],
                ssems[k % 2],
            )

        load(0).start()
        for k in range(nchunks):
            load(k).wait()
            store(k).start()
            if k + 1 < nchunks:
                if k >= 1:
                    store(k - 1).wait()
                load(k + 1).start()
        if nchunks >= 2:
            store(nchunks - 2).wait()
        store(nchunks - 1).wait()

    return sc_copy(x)


def kernel(node_latents_from, node_latents_to, edge_latents, edge_index, receivers_count):
    node_copy = _tc_copy(node_latents_to)
    edge_copy = _sc_copy(edge_latents)
    if edge_copy is None:
        edge_copy = _tc_copy(edge_latents)
    return (node_copy, edge_copy)


# speedup vs baseline: 9.9450x; 9.9450x over previous
import jax
import jax.numpy as jnp
from jax.experimental import pallas as pl


def _copy_body(src, dst):
    dst[...] = src[...]


def _tc_copy(x, max_grid=40):
    rows, cols = x.shape
    g = 1
    for cand in range(max_grid, 0, -1):
        if rows % cand == 0 and (rows // cand) % 8 == 0:
            g = cand
            break
    b = rows // g
    return pl.pallas_call(
        _copy_body,
        grid=(g,),
        in_specs=(pl.BlockSpec((b, cols), lambda i: (i, 0)),),
        out_specs=pl.BlockSpec((b, cols), lambda i: (i, 0)),
        out_shape=jax.ShapeDtypeStruct((rows, cols), x.dtype),
    )(x)


def kernel(node_latents_from, node_latents_to, edge_latents, edge_index, receivers_count):
    new_node_latents = _tc_copy(node_latents_to)
    return (new_node_latents, edge_latents)


# speedup vs baseline: 9.9939x; 1.0049x over previous
import jax
import jax.numpy as jnp
from jax.experimental import pallas as pl


def _copy_body(src, dst):
    dst[...] = src[...]


def _tc_copy(x, max_grid=25):
    rows, cols = x.shape
    g = 1
    for cand in range(max_grid, 0, -1):
        if rows % cand == 0 and (rows // cand) % 8 == 0:
            g = cand
            break
    b = rows // g
    return pl.pallas_call(
        _copy_body,
        grid=(g,),
        in_specs=(pl.BlockSpec((b, cols), lambda i: (i, 0)),),
        out_specs=pl.BlockSpec((b, cols), lambda i: (i, 0)),
        out_shape=jax.ShapeDtypeStruct((rows, cols), x.dtype),
    )(x)


def kernel(node_latents_from, node_latents_to, edge_latents, edge_index, receivers_count):
    new_node_latents = _tc_copy(node_latents_to)
    return (new_node_latents, edge_latents)


# speedup vs baseline: 14.0689x; 1.4078x over previous
import jax
import jax.numpy as jnp
from jax.experimental import pallas as pl


def _copy_body(src, dst):
    dst[...] = src[...]


def _tc_copy(x, max_grid=8):
    rows, cols = x.shape
    g = 1
    for cand in range(max_grid, 0, -1):
        if rows % cand == 0 and (rows // cand) % 8 == 0:
            g = cand
            break
    b = rows // g
    return pl.pallas_call(
        _copy_body,
        grid=(g,),
        in_specs=(pl.BlockSpec((b, cols), lambda i: (i, 0)),),
        out_specs=pl.BlockSpec((b, cols), lambda i: (i, 0)),
        out_shape=jax.ShapeDtypeStruct((rows, cols), x.dtype),
    )(x)


def kernel(node_latents_from, node_latents_to, edge_latents, edge_index, receivers_count):
    new_node_latents = _tc_copy(node_latents_to)
    return (new_node_latents, edge_latents)


# speedup vs baseline: 14.1128x; 1.0031x over previous
import jax
import jax.numpy as jnp
from jax.experimental import pallas as pl


def _copy_body(src, dst):
    dst[...] = src[...]


def _tc_copy(x, max_grid=5):
    rows, cols = x.shape
    g = 1
    for cand in range(max_grid, 0, -1):
        if rows % cand == 0 and (rows // cand) % 8 == 0:
            g = cand
            break
    b = rows // g
    return pl.pallas_call(
        _copy_body,
        grid=(g,),
        in_specs=(pl.BlockSpec((b, cols), lambda i: (i, 0)),),
        out_specs=pl.BlockSpec((b, cols), lambda i: (i, 0)),
        out_shape=jax.ShapeDtypeStruct((rows, cols), x.dtype),
    )(x)


def kernel(node_latents_from, node_latents_to, edge_latents, edge_index, receivers_count):
    new_node_latents = _tc_copy(node_latents_to)
    return (new_node_latents, edge_latents)


# speedup vs baseline: 15.8076x; 1.1201x over previous
import jax
import jax.numpy as jnp
from jax.experimental import pallas as pl


def _copy_body(src, dst):
    dst[...] = src[...]


def _tc_copy(x, max_grid=2):
    rows, cols = x.shape
    g = 1
    for cand in range(max_grid, 0, -1):
        if rows % cand == 0 and (rows // cand) % 8 == 0:
            g = cand
            break
    b = rows // g
    return pl.pallas_call(
        _copy_body,
        grid=(g,),
        in_specs=(pl.BlockSpec((b, cols), lambda i: (i, 0)),),
        out_specs=pl.BlockSpec((b, cols), lambda i: (i, 0)),
        out_shape=jax.ShapeDtypeStruct((rows, cols), x.dtype),
    )(x)


def kernel(node_latents_from, node_latents_to, edge_latents, edge_index, receivers_count):
    new_node_latents = _tc_copy(node_latents_to)
    return (new_node_latents, edge_latents)
